# baseline (device time: 152862 ns/iter reference)
import jax
import jax.numpy as jnp
from jax import lax
from jax.experimental import pallas as pl
from jax.experimental.pallas import tpu as pltpu

N_DEV = 32
LOG2 = 5
B, SQ, SKV, HQ_LOC, DH = 2, 256, 256, 4, 64
D_MODEL = 512
HD = HQ_LOC * DH
ROWS = B * SQ
CHUNK = ROWS // N_DEV

RS_HALF = [ROWS >> (k + 1) for k in range(LOG2)]
RS_OFF = [0]
for _h in RS_HALF[:-1]:
    RS_OFF.append(RS_OFF[-1] + _h)
RS_BUF_ROWS = sum(RS_HALF)


def kernel(x, Wq, K_ext, V_ext, Wo):
    x2 = x.reshape(ROWS, D_MODEL)
    K2 = K_ext.reshape(B * SKV, 128 * DH)
    V2 = V_ext.reshape(B * SKV, 128 * DH)

    def body(x_ref, wq_ref, k_hbm, v_hbm, wo_ref, out_ref,
             kbuf, vbuf, ctx_ref, acc_ref, rs_buf,
             copy_sems, rs_send, rs_recv, ag_send, ag_recv):
        my = lax.axis_index("i")

        col = pl.multiple_of(my * HD, HD)
        k_copy = pltpu.make_async_copy(
            k_hbm.at[:, pl.ds(col, HD)], kbuf, copy_sems.at[0])
        v_copy = pltpu.make_async_copy(
            v_hbm.at[:, pl.ds(col, HD)], vbuf, copy_sems.at[1])
        k_copy.start()
        v_copy.start()

        q = jnp.dot(x_ref[:, :], wq_ref[:, :], preferred_element_type=jnp.float32)
        k_copy.wait()
        v_copy.wait()

        for b in range(B):
            for h in range(HQ_LOC):
                qbh = q[b * SQ:(b + 1) * SQ, h * DH:(h + 1) * DH]
                kbh = kbuf[b * SKV:(b + 1) * SKV, h * DH:(h + 1) * DH]
                s = lax.dot_general(
                    qbh, kbh, (((1,), (1,)), ((), ())),
                    preferred_element_type=jnp.float32,
                ) * 0.125
                rb = lax.broadcasted_iota(jnp.int32, (SQ, SKV), 0) // 64
                cb = lax.broadcasted_iota(jnp.int32, (SQ, SKV), 1) // 64
                s = jnp.where(cb <= rb, s, -1e9)
                m = jnp.max(s, axis=1, keepdims=True)
                e = jnp.exp(s - m)
                w = e / jnp.sum(e, axis=1, keepdims=True)
                ctx_ref[b * SQ:(b + 1) * SQ, h * DH:(h + 1) * DH] = jnp.dot(
                    w, vbuf[b * SKV:(b + 1) * SKV, h * DH:(h + 1) * DH],
                    preferred_element_type=jnp.float32)
        acc_ref[:, :] = jnp.dot(ctx_ref[:, :], wo_ref[:, :],
                                preferred_element_type=jnp.float32)

        bar = pltpu.get_barrier_semaphore()
        for k in range(LOG2):
            pl.semaphore_signal(bar, inc=1, device_id=(my ^ (1 << k),),
                                device_id_type=pl.DeviceIdType.MESH)
        pl.semaphore_wait(bar, LOG2)

        lo = my * 0
        for k in range(LOG2):
            half = RS_HALF[k]
            bit = (my >> k) & 1
            keep_lo = pl.multiple_of(lo + bit * half, CHUNK)
            send_lo = pl.multiple_of(lo + (1 - bit) * half, CHUNK)
            rdma = pltpu.make_async_remote_copy(
                src_ref=acc_ref.at[pl.ds(send_lo, half), :],
                dst_ref=rs_buf.at[pl.ds(RS_OFF[k], half), :],
                send_sem=rs_send.at[k],
                recv_sem=rs_recv.at[k],
                device_id=(my ^ (1 << k),),
                device_id_type=pl.DeviceIdType.MESH,
            )
            rdma.start()
            rdma.wait()
            acc_ref[pl.ds(keep_lo, half), :] = (
                acc_ref[pl.ds(keep_lo, half), :]
                + rs_buf[pl.ds(RS_OFF[k], half), :]
            )
            lo = keep_lo
        out_ref[pl.ds(lo, CHUNK), :] = acc_ref[pl.ds(lo, CHUNK), :]

        for idx, j in enumerate(range(LOG2 - 1, -1, -1)):
            size = CHUNK << (LOG2 - 1 - j)
            glo = pl.multiple_of(lo & ~(size - 1), size)
            rdma = pltpu.make_async_remote_copy(
                src_ref=out_ref.at[pl.ds(glo, size), :],
                dst_ref=out_ref.at[pl.ds(glo, size), :],
                send_sem=ag_send.at[idx],
                recv_sem=ag_recv.at[idx],
                device_id=(my ^ (1 << j),),
                device_id_type=pl.DeviceIdType.MESH,
            )
            rdma.start()
            rdma.wait()

    out = pl.pallas_call(
        body,
        out_shape=jax.ShapeDtypeStruct((ROWS, D_MODEL), jnp.float32),
        in_specs=[
            pl.BlockSpec(memory_space=pltpu.VMEM),
            pl.BlockSpec(memory_space=pltpu.VMEM),
            pl.BlockSpec(memory_space=pltpu.HBM),
            pl.BlockSpec(memory_space=pltpu.HBM),
            pl.BlockSpec(memory_space=pltpu.VMEM),
        ],
        out_specs=pl.BlockSpec(memory_space=pltpu.VMEM),
        scratch_shapes=[
            pltpu.VMEM((B * SKV, HD), jnp.float32),
            pltpu.VMEM((B * SKV, HD), jnp.float32),
            pltpu.VMEM((ROWS, HD), jnp.float32),
            pltpu.VMEM((ROWS, D_MODEL), jnp.float32),
            pltpu.VMEM((RS_BUF_ROWS, D_MODEL), jnp.float32),
            pltpu.SemaphoreType.DMA((2,)),
            pltpu.SemaphoreType.DMA((LOG2,)),
            pltpu.SemaphoreType.DMA((LOG2,)),
            pltpu.SemaphoreType.DMA((LOG2,)),
            pltpu.SemaphoreType.DMA((LOG2,)),
        ],
        compiler_params=pltpu.CompilerParams(collective_id=0),
    )(x2, Wq, K2, V2, Wo)
    return out.reshape(B, SQ, D_MODEL)


# device time: 87789 ns/iter; 1.7412x vs baseline; 1.7412x over previous
import jax
import jax.numpy as jnp
from jax import lax
from jax.experimental import pallas as pl
from jax.experimental.pallas import tpu as pltpu

N_DEV = 32
LOG2 = 5
B, SQ, SKV, HQ_LOC, DH = 2, 256, 256, 4, 64
D_MODEL = 512
HD = HQ_LOC * DH
ROWS = B * SQ
CHUNK = ROWS // N_DEV

RS_HALF = [ROWS >> (k + 1) for k in range(LOG2)]
RS_OFF = [0]
for _h in RS_HALF[:-1]:
    RS_OFF.append(RS_OFF[-1] + _h)
RS_BUF_ROWS = sum(RS_HALF)


def kernel(x, Wq, K_ext, V_ext, Wo):
    x2 = x.reshape(ROWS, D_MODEL)
    i = lax.axis_index("i")
    K_loc = lax.dynamic_slice(
        K_ext, (0, 0, i * HQ_LOC, 0), (B, SKV, HQ_LOC, DH)).reshape(B * SKV, HD)
    V_loc = lax.dynamic_slice(
        V_ext, (0, 0, i * HQ_LOC, 0), (B, SKV, HQ_LOC, DH)).reshape(B * SKV, HD)

    def body(x_ref, wq_ref, kbuf, vbuf, wo_ref, out_ref,
             ctx_ref, acc_ref, rs_buf,
             rs_send, rs_recv, ag_send, ag_recv):
        my = lax.axis_index("i")

        q = jnp.dot(x_ref[:, :], wq_ref[:, :], preferred_element_type=jnp.float32)

        for b in range(B):
            for h in range(HQ_LOC):
                qbh = q[b * SQ:(b + 1) * SQ, h * DH:(h + 1) * DH]
                kbh = kbuf[b * SKV:(b + 1) * SKV, h * DH:(h + 1) * DH]
                s = lax.dot_general(
                    qbh, kbh, (((1,), (1,)), ((), ())),
                    preferred_element_type=jnp.float32,
                ) * 0.125
                rb = lax.broadcasted_iota(jnp.int32, (SQ, SKV), 0) // 64
                cb = lax.broadcasted_iota(jnp.int32, (SQ, SKV), 1) // 64
                s = jnp.where(cb <= rb, s, -1e9)
                m = jnp.max(s, axis=1, keepdims=True)
                e = jnp.exp(s - m)
                w = e / jnp.sum(e, axis=1, keepdims=True)
                ctx_ref[b * SQ:(b + 1) * SQ, h * DH:(h + 1) * DH] = jnp.dot(
                    w, vbuf[b * SKV:(b + 1) * SKV, h * DH:(h + 1) * DH],
                    preferred_element_type=jnp.float32)
        acc_ref[:, :] = jnp.dot(ctx_ref[:, :], wo_ref[:, :],
                                preferred_element_type=jnp.float32)

        bar = pltpu.get_barrier_semaphore()
        for k in range(LOG2):
            pl.semaphore_signal(bar, inc=1, device_id=(my ^ (1 << k),),
                                device_id_type=pl.DeviceIdType.MESH)
        pl.semaphore_wait(bar, LOG2)

        lo = my * 0
        for k in range(LOG2):
            half = RS_HALF[k]
            bit = (my >> k) & 1
            keep_lo = pl.multiple_of(lo + bit * half, CHUNK)
            send_lo = pl.multiple_of(lo + (1 - bit) * half, CHUNK)
            rdma = pltpu.make_async_remote_copy(
                src_ref=acc_ref.at[pl.ds(send_lo, half), :],
                dst_ref=rs_buf.at[pl.ds(RS_OFF[k], half), :],
                send_sem=rs_send.at[k],
                recv_sem=rs_recv.at[k],
                device_id=(my ^ (1 << k),),
                device_id_type=pl.DeviceIdType.MESH,
            )
            rdma.start()
            rdma.wait()
            acc_ref[pl.ds(keep_lo, half), :] = (
                acc_ref[pl.ds(keep_lo, half), :]
                + rs_buf[pl.ds(RS_OFF[k], half), :]
            )
            lo = keep_lo
        out_ref[pl.ds(lo, CHUNK), :] = acc_ref[pl.ds(lo, CHUNK), :]

        for idx, j in enumerate(range(LOG2 - 1, -1, -1)):
            size = CHUNK << (LOG2 - 1 - j)
            glo = pl.multiple_of(lo & ~(size - 1), size)
            rdma = pltpu.make_async_remote_copy(
                src_ref=out_ref.at[pl.ds(glo, size), :],
                dst_ref=out_ref.at[pl.ds(glo, size), :],
                send_sem=ag_send.at[idx],
                recv_sem=ag_recv.at[idx],
                device_id=(my ^ (1 << j),),
                device_id_type=pl.DeviceIdType.MESH,
            )
            rdma.start()
            rdma.wait()

    out = pl.pallas_call(
        body,
        out_shape=jax.ShapeDtypeStruct((ROWS, D_MODEL), jnp.float32),
        in_specs=[pl.BlockSpec(memory_space=pltpu.VMEM)] * 5,
        out_specs=pl.BlockSpec(memory_space=pltpu.VMEM),
        scratch_shapes=[
            pltpu.VMEM((ROWS, HD), jnp.float32),
            pltpu.VMEM((ROWS, D_MODEL), jnp.float32),
            pltpu.VMEM((RS_BUF_ROWS, D_MODEL), jnp.float32),
            pltpu.SemaphoreType.DMA((LOG2,)),
            pltpu.SemaphoreType.DMA((LOG2,)),
            pltpu.SemaphoreType.DMA((LOG2,)),
            pltpu.SemaphoreType.DMA((LOG2,)),
        ],
        compiler_params=pltpu.CompilerParams(collective_id=0),
    )(x2, Wq, K_loc, V_loc, Wo)
    return out.reshape(B, SQ, D_MODEL)
